# baseline (speedup 1.0000x reference)
import functools

import jax
import jax.numpy as jnp
from jax import lax
from jax.experimental import pallas as pl
from jax.experimental.pallas import tpu as pltpu
from jax.experimental.pallas import tpu_sc as plsc

L = 2048
NCORES = 2
NSUB = 16
NW = NCORES * NSUB
ROWS_PER_W = L // NW
BLK = 8
NBLK = ROWS_PER_W // BLK
NBUF = 3
LANES = 16
NCH = L // LANES

PARTNER_LUT = 0x4A52


def _body(con_hbm, feat_hbm, out_hbm, idx_v, gseq_v, codes_v, rowp_v,
          in_v, out_v, shared_cr, sem_in, sem_out, sem_g):
    cid = lax.axis_index("c")
    sid = lax.axis_index("s")
    wid = sid * NCORES + cid
    row0 = wid * ROWS_PER_W

    def in_copy(blk, buf):
        return pltpu.make_async_copy(
            con_hbm.at[pl.ds(row0 + blk * BLK, BLK)], in_v.at[buf], sem_in)

    def out_copy(blk, buf):
        return pltpu.make_async_copy(
            out_v.at[buf], out_hbm.at[pl.ds(row0 + blk * BLK, BLK)], sem_out)

    for k in range(NBUF):
        in_copy(k, k).start()

    lane = lax.iota(jnp.int32, LANES)

    @pl.when(sid == 0)
    def _stage():
        for c in range(4):
            @plsc.parallel_loop(0, NCH, unroll=2)
            def _mk_idx(i):
                sl = pl.ds(c * L + i * LANES, LANES)
                idx_v[sl] = (c * L * L) + (i * LANES + lane) * L

        pltpu.async_copy(feat_hbm.at[idx_v], gseq_v, sem_g).wait()

        @plsc.parallel_loop(0, NCH, unroll=2)
        def _class_chunk(i):
            v = gseq_v[pl.ds(i * LANES, LANES)]
            cc = jnp.zeros((LANES,), jnp.int32)
            for k in (1, 2, 3):
                s = gseq_v[pl.ds(k * L + i * LANES, LANES)]
                upd = s > v
                cc = jnp.where(upd, k, cc)
                v = jnp.maximum(v, s)
            sl = pl.ds(i * LANES, LANES)
            codes_v[sl] = jnp.int32(1) << cc
            rowp_v[sl] = (jnp.int32(PARTNER_LUT) >> (cc * 4)) & 0xF

        pltpu.sync_copy(codes_v, shared_cr.at[0])
        pltpu.sync_copy(rowp_v.at[pl.ds(0, L)], shared_cr.at[1])

    plsc.subcore_barrier()

    @pl.when(sid != 0)
    def _fetch():
        pltpu.sync_copy(shared_cr.at[0], codes_v)
        pltpu.sync_copy(shared_cr.at[1], rowp_v.at[pl.ds(0, L)])

    ftab = jnp.where(lane == 0, 0.0, 1.0).astype(jnp.float32)

    def block(blk, _):
        b = lax.rem(blk, NBUF)
        in_copy(blk, b).wait()

        @pl.when(blk >= NBUF)
        def _():
            out_copy(blk - NBUF, b).wait()

        rowp16 = rowp_v[pl.ds(row0 + blk * BLK, LANES)]
        pvecs = [
            rowp16.at[jnp.full((LANES,), rr, jnp.int32)].get(
                mode="promise_in_bounds")
            for rr in range(BLK)
        ]

        @plsc.parallel_loop(0, NCH, unroll=2)
        def _cols(ci):
            sl = pl.ds(ci * LANES, LANES)
            code = codes_v[sl]
            for rr in range(BLK):
                x = code & pvecs[rr]
                fm = ftab.at[x].get(mode="promise_in_bounds")
                out_v[b, rr, sl] = in_v[b, rr, sl] * fm

        out_copy(blk, b).start()

        @pl.when(blk + NBUF < NBLK)
        def _():
            in_copy(blk + NBUF, b).start()

        return 0

    lax.fori_loop(0, NBLK, block, 0)
    for k in range(NBUF):
        blk = NBLK - NBUF + k
        out_copy(blk, lax.rem(jnp.int32(blk), NBUF)).wait()


@jax.jit
def _canonicalize(con2d, feat_flat):
    mesh = plsc.VectorSubcoreMesh(core_axis_name="c", subcore_axis_name="s")
    f = functools.partial(
        pl.kernel,
        mesh=mesh,
        out_type=jax.ShapeDtypeStruct((L, L), jnp.float32),
        scratch_types=[
            pltpu.VMEM((4 * L,), jnp.int32),
            pltpu.VMEM((4 * L,), jnp.float32),
            pltpu.VMEM((L,), jnp.int32),
            pltpu.VMEM((L + LANES,), jnp.int32),
            pltpu.VMEM((NBUF, BLK, L), jnp.float32),
            pltpu.VMEM((NBUF, BLK, L), jnp.float32),
            pltpu.VMEM_SHARED((2, L), jnp.int32),
            pltpu.SemaphoreType.DMA,
            pltpu.SemaphoreType.DMA,
            pltpu.SemaphoreType.DMA,
        ],
    )(_body)
    return f(con2d, feat_flat)


def kernel(con, feat):
    con2d = con.reshape(L, L)
    feat_flat = feat.reshape(-1)
    out = _canonicalize(con2d, feat_flat)
    return out.reshape(con.shape)


# speedup vs baseline: 3.6784x; 3.6784x over previous
import functools

import jax
import jax.numpy as jnp
from jax import lax
from jax.experimental import pallas as pl
from jax.experimental.pallas import tpu as pltpu
from jax.experimental.pallas import tpu_sc as plsc

L = 2048
NCORES = 2
NSUB = 16
NW = NCORES * NSUB
ROWS_PER_W = L // NW
BLK = 8
NBLK = ROWS_PER_W // BLK
NBUF = 3
LANES = 16
NCH = L // LANES

PARTNER_LUT = 0x4A52


def _body(con_hbm, s0_hbm, s1_hbm, s2_hbm, s3_hbm, out_hbm, seq_v, codes_v,
          rowp_v, in_v, out_v, sem_in, sem_out):
    wid = lax.axis_index("s") * NCORES + lax.axis_index("c")
    row0 = wid * ROWS_PER_W

    def in_copy(blk, buf):
        return pltpu.make_async_copy(
            con_hbm.at[pl.ds(row0 + blk * BLK, BLK)], in_v.at[buf], sem_in)

    def out_copy(blk, buf):
        return pltpu.make_async_copy(
            out_v.at[buf], out_hbm.at[pl.ds(row0 + blk * BLK, BLK)], sem_out)

    for k in range(NBUF):
        in_copy(k, k).start()

    for c, s_hbm in enumerate((s0_hbm, s1_hbm, s2_hbm, s3_hbm)):
        pltpu.sync_copy(s_hbm, seq_v.at[c])

    @plsc.parallel_loop(0, NCH, unroll=2)
    def _class_chunk(i):
        sl = pl.ds(i * LANES, LANES)
        v = seq_v[0, sl]
        c = jnp.zeros((LANES,), jnp.int32)
        for k in (1, 2, 3):
            s = seq_v[k, sl]
            upd = s > v
            c = jnp.where(upd, k, c)
            v = jnp.maximum(v, s)
        codes_v[sl] = jnp.int32(1) << c
        rowp_v[sl] = (jnp.int32(PARTNER_LUT) >> (c * 4)) & 0xF

    idx16 = lax.iota(jnp.int32, LANES)
    ftab = jnp.where(idx16 == 0, 0.0, 1.0).astype(jnp.float32)

    def block(blk, _):
        b = lax.rem(blk, NBUF)
        in_copy(blk, b).wait()

        @pl.when(blk >= NBUF)
        def _():
            out_copy(blk - NBUF, b).wait()

        rowp16 = rowp_v[pl.ds(row0 + blk * BLK, LANES)]
        pvecs = [
            rowp16.at[jnp.full((LANES,), rr, jnp.int32)].get(
                mode="promise_in_bounds")
            for rr in range(BLK)
        ]

        @plsc.parallel_loop(0, NCH, unroll=2)
        def _cols(ci):
            sl = pl.ds(ci * LANES, LANES)
            code = codes_v[sl]
            for rr in range(BLK):
                x = code & pvecs[rr]
                fm = ftab.at[x].get(mode="promise_in_bounds")
                out_v[b, rr, sl] = in_v[b, rr, sl] * fm

        out_copy(blk, b).start()

        @pl.when(blk + NBUF < NBLK)
        def _():
            in_copy(blk + NBUF, b).start()

        return 0

    lax.fori_loop(0, NBLK, block, 0)
    for k in range(NBUF):
        blk = NBLK - NBUF + k
        out_copy(blk, lax.rem(jnp.int32(blk), NBUF)).wait()


@jax.jit
def _canonicalize(con2d, s0, s1, s2, s3):
    mesh = plsc.VectorSubcoreMesh(core_axis_name="c", subcore_axis_name="s")
    f = functools.partial(
        pl.kernel,
        mesh=mesh,
        out_type=jax.ShapeDtypeStruct((L, L), jnp.float32),
        scratch_types=[
            pltpu.VMEM((4, L), jnp.float32),
            pltpu.VMEM((L,), jnp.int32),
            pltpu.VMEM((L + LANES,), jnp.int32),
            pltpu.VMEM((NBUF, BLK, L), jnp.float32),
            pltpu.VMEM((NBUF, BLK, L), jnp.float32),
            pltpu.SemaphoreType.DMA,
            pltpu.SemaphoreType.DMA,
        ],
    )(_body)
    return f(con2d, s0, s1, s2, s3)


def kernel(con, feat):
    con2d = con.reshape(L, L)
    s0, s1, s2, s3 = (feat[0, c, :, 0] for c in range(4))
    out = _canonicalize(con2d, s0, s1, s2, s3)
    return out.reshape(con.shape)


# speedup vs baseline: 3.9103x; 1.0631x over previous
import functools

import jax
import jax.numpy as jnp
from jax import lax
from jax.experimental import pallas as pl
from jax.experimental.pallas import tpu as pltpu
from jax.experimental.pallas import tpu_sc as plsc

L = 2048
NCORES = 2
NSUB = 16
NW = NCORES * NSUB
ROWS_PER_W = L // NW
BLK = 8
NBLK = ROWS_PER_W // BLK
NBUF = 3
LANES = 16
NCH = L // LANES

PARTNER_LUT = 0x4A52


def _body(con_hbm, seq_hbm, out_hbm, seq_v, codes_v, rowp_v, in_v, out_v,
          sem_in, sem_out):
    wid = lax.axis_index("s") * NCORES + lax.axis_index("c")
    row0 = wid * ROWS_PER_W

    def in_copy(blk, buf):
        return pltpu.make_async_copy(
            con_hbm.at[pl.ds(row0 + blk * BLK, BLK)], in_v.at[buf], sem_in)

    def out_copy(blk, buf):
        return pltpu.make_async_copy(
            out_v.at[buf], out_hbm.at[pl.ds(row0 + blk * BLK, BLK)], sem_out)

    for k in range(NBUF):
        in_copy(k, k).start()

    pltpu.sync_copy(seq_hbm, seq_v)

    @plsc.parallel_loop(0, NCH, unroll=2)
    def _class_chunk(i):
        sl = pl.ds(i * LANES, LANES)
        v = seq_v[0, sl]
        c = jnp.zeros((LANES,), jnp.int32)
        for k in (1, 2, 3):
            s = seq_v[k, sl]
            upd = s > v
            c = jnp.where(upd, k, c)
            v = jnp.maximum(v, s)
        codes_v[sl] = jnp.int32(1) << c
        rowp_v[sl] = (jnp.int32(PARTNER_LUT) >> (c * 4)) & 0xF

    idx16 = lax.iota(jnp.int32, LANES)
    ftab = jnp.where(idx16 == 0, 0.0, 1.0).astype(jnp.float32)

    def block(blk, _):
        b = lax.rem(blk, NBUF)
        in_copy(blk, b).wait()

        @pl.when(blk >= NBUF)
        def _():
            out_copy(blk - NBUF, b).wait()

        rowp16 = rowp_v[pl.ds(row0 + blk * BLK, LANES)]
        pvecs = [
            rowp16.at[jnp.full((LANES,), rr, jnp.int32)].get(
                mode="promise_in_bounds")
            for rr in range(BLK)
        ]

        @plsc.parallel_loop(0, NCH, unroll=2)
        def _cols(ci):
            sl = pl.ds(ci * LANES, LANES)
            code = codes_v[sl]
            for rr in range(BLK):
                x = code & pvecs[rr]
                fm = ftab.at[x].get(mode="promise_in_bounds")
                out_v[b, rr, sl] = in_v[b, rr, sl] * fm

        out_copy(blk, b).start()

        @pl.when(blk + NBUF < NBLK)
        def _():
            in_copy(blk + NBUF, b).start()

        return 0

    lax.fori_loop(0, NBLK, block, 0)
    for k in range(NBUF):
        blk = NBLK - NBUF + k
        out_copy(blk, lax.rem(jnp.int32(blk), NBUF)).wait()


@jax.jit
def _canonicalize(con2d, seq):
    mesh = plsc.VectorSubcoreMesh(core_axis_name="c", subcore_axis_name="s")
    f = functools.partial(
        pl.kernel,
        mesh=mesh,
        out_type=jax.ShapeDtypeStruct((L, L), jnp.float32),
        scratch_types=[
            pltpu.VMEM((4, L), jnp.float32),
            pltpu.VMEM((L,), jnp.int32),
            pltpu.VMEM((L + LANES,), jnp.int32),
            pltpu.VMEM((NBUF, BLK, L), jnp.float32),
            pltpu.VMEM((NBUF, BLK, L), jnp.float32),
            pltpu.SemaphoreType.DMA,
            pltpu.SemaphoreType.DMA,
        ],
    )(_body)
    return f(con2d, seq)


def kernel(con, feat):
    con2d = con.reshape(L, L)
    seq = feat[0, :4, :, 0]
    out = _canonicalize(con2d, seq)
    return out.reshape(con.shape)


# speedup vs baseline: 3.9653x; 1.0141x over previous
import functools

import jax
import jax.numpy as jnp
from jax import lax
from jax.experimental import pallas as pl
from jax.experimental.pallas import tpu as pltpu
from jax.experimental.pallas import tpu_sc as plsc

L = 2048
NCORES = 2
NSUB = 16
NW = NCORES * NSUB
ROWS_PER_W = L // NW
BLK = 8
NBLK = ROWS_PER_W // BLK
NBUF = 3
LANES = 16
NCH = L // LANES

PARTNER_LUT = 0x4A52


def _body(con_hbm, seq_hbm, out_hbm, seq_v, codes_v, rowp_v, in_v, out_v,
          sem_in, sem_out):
    wid = lax.axis_index("s") * NCORES + lax.axis_index("c")
    row0 = wid * ROWS_PER_W

    def in_copy(blk, buf):
        return pltpu.make_async_copy(
            con_hbm.at[pl.ds(row0 + blk * BLK, BLK)], in_v.at[buf], sem_in)

    def out_copy(blk, buf):
        return pltpu.make_async_copy(
            out_v.at[buf], out_hbm.at[pl.ds(row0 + blk * BLK, BLK)], sem_out)

    for k in range(NBUF):
        in_copy(k, k).start()

    pltpu.sync_copy(seq_hbm, seq_v)

    @plsc.parallel_loop(0, NCH, unroll=2)
    def _class_chunk(i):
        sl = pl.ds(i * LANES, LANES)
        v = seq_v[0, sl]
        c = jnp.zeros((LANES,), jnp.int32)
        for k in (1, 2, 3):
            s = seq_v[k, sl]
            upd = s > v
            c = jnp.where(upd, k, c)
            v = jnp.maximum(v, s)
        codes_v[sl] = jnp.int32(1) << c
        rowp_v[sl] = (jnp.int32(PARTNER_LUT) >> (c * 4)) & 0xF

    idx16 = lax.iota(jnp.int32, LANES)
    ftab = jnp.where(idx16 == 0, 0.0, 1.0).astype(jnp.float32)

    def block(blk, _):
        b = lax.rem(blk, NBUF)
        in_copy(blk, b).wait()

        @pl.when(blk >= NBUF)
        def _():
            out_copy(blk - NBUF, b).wait()

        rowp16 = rowp_v[pl.ds(row0 + blk * BLK, LANES)]
        pvecs = [
            rowp16.at[jnp.full((LANES,), rr, jnp.int32)].get(
                mode="promise_in_bounds")
            for rr in range(BLK)
        ]

        @plsc.parallel_loop(0, NCH, unroll=4)
        def _cols(ci):
            sl = pl.ds(ci * LANES, LANES)
            code = codes_v[sl]
            for rr in range(BLK):
                x = code & pvecs[rr]
                fm = ftab.at[x].get(mode="promise_in_bounds")
                out_v[b, rr, sl] = in_v[b, rr, sl] * fm

        out_copy(blk, b).start()

        @pl.when(blk + NBUF < NBLK)
        def _():
            in_copy(blk + NBUF, b).start()

        return 0

    lax.fori_loop(0, NBLK, block, 0)
    for k in range(NBUF):
        blk = NBLK - NBUF + k
        out_copy(blk, lax.rem(jnp.int32(blk), NBUF)).wait()


@jax.jit
def _canonicalize(con2d, seq):
    mesh = plsc.VectorSubcoreMesh(core_axis_name="c", subcore_axis_name="s")
    f = functools.partial(
        pl.kernel,
        mesh=mesh,
        out_type=jax.ShapeDtypeStruct((L, L), jnp.float32),
        scratch_types=[
            pltpu.VMEM((4, L), jnp.float32),
            pltpu.VMEM((L,), jnp.int32),
            pltpu.VMEM((L + LANES,), jnp.int32),
            pltpu.VMEM((NBUF, BLK, L), jnp.float32),
            pltpu.VMEM((NBUF, BLK, L), jnp.float32),
            pltpu.SemaphoreType.DMA,
            pltpu.SemaphoreType.DMA,
        ],
    )(_body)
    return f(con2d, seq)


def kernel(con, feat):
    con2d = con.reshape(L, L)
    seq = feat[0, :4, :, 0]
    out = _canonicalize(con2d, seq)
    return out.reshape(con.shape)
